# baseline (speedup 1.0000x reference)
import functools

import jax
import jax.numpy as jnp
from jax import lax
from jax.experimental import pallas as pl
from jax.experimental.pallas import tpu as pltpu
from jax.experimental.pallas import tpu_sc as plsc

NC = 2
NS = 16
NW = NC * NS
EB = 128


def _sc_mesh():
    return plsc.VectorSubcoreMesh(core_axis_name="c", subcore_axis_name="s")


def _deg_kernel(n_pad, iters, iters_pad, d):
    rt = n_pad // NS
    fire = 8
    rounds = iters // fire
    tail = iters - rounds * fire

    @functools.partial(
        pl.kernel,
        out_type=jax.ShapeDtypeStruct((NC, n_pad, d), jnp.float32),
        mesh=_sc_mesh(),
        scratch_types=[
            pltpu.VMEM((iters_pad, EB), jnp.int32),
            pltpu.VMEM((EB, d), jnp.float32),
            pltpu.SemaphoreType.DMA,
            pltpu.VMEM_SHARED((n_pad, d), jnp.float32),
        ],
    )
    def k(dst_hbm, ones_hbm, zeros_hbm, out_hbm, idx_v, ones_v, sem, acc_s):
        cid = lax.axis_index("c")
        sid = lax.axis_index("s")
        wid = sid * NC + cid
        pltpu.sync_copy(dst_hbm.at[wid], idx_v)
        pltpu.sync_copy(ones_hbm, ones_v)
        pltpu.sync_copy(zeros_hbm, acc_s.at[pl.ds(sid * rt, rt)])
        plsc.subcore_barrier()

        def body(r, carry):
            for j in range(fire):
                pltpu.async_copy(ones_v, acc_s.at[idx_v.at[r * fire + j]],
                                 sem, add=True)
            for j in range(fire):
                pltpu.make_async_copy(
                    ones_v, acc_s.at[idx_v.at[r * fire + j]], sem).wait()
            return carry

        lax.fori_loop(0, rounds, body, 0)
        for j in range(tail):
            pltpu.async_copy(ones_v, acc_s.at[idx_v.at[rounds * fire + j]],
                             sem, add=True)
        for j in range(tail):
            pltpu.make_async_copy(
                ones_v, acc_s.at[idx_v.at[rounds * fire + j]], sem).wait()
        plsc.subcore_barrier()
        pltpu.sync_copy(acc_s.at[pl.ds(sid * rt, rt)],
                        out_hbm.at[cid, pl.ds(sid * rt, rt)])

    return k


def _agg_kernel(n_pad, iters, iters_pad, d):
    rt = n_pad // NS
    t_steps = iters // 3

    @functools.partial(
        pl.kernel,
        out_type=jax.ShapeDtypeStruct((NC, n_pad, d), jnp.float32),
        mesh=_sc_mesh(),
        scratch_types=[
            pltpu.VMEM((3, EB), jnp.int32),
            pltpu.VMEM((3, EB), jnp.int32),
            pltpu.VMEM((EB, d), jnp.float32),
            pltpu.VMEM((EB, d), jnp.float32),
            pltpu.VMEM((EB, d), jnp.float32),
            pltpu.VMEM_SHARED((n_pad, d), jnp.float32),
            pltpu.SemaphoreType.DMA,
            pltpu.SemaphoreType.DMA,
            pltpu.SemaphoreType.DMA,
            pltpu.SemaphoreType.DMA,
            pltpu.SemaphoreType.DMA,
            pltpu.SemaphoreType.DMA,
        ],
    )
    def k(xs_hbm, src_hbm, dst_hbm, zeros_hbm, out_hbm,
          srcr, dstr, rows_a, rows_b, rows_c, acc_s,
          sg0, sg1, sg2, si0, si1, si2):
        cid = lax.axis_index("c")
        sid = lax.axis_index("s")
        wid = sid * NC + cid
        rows = [rows_a, rows_b, rows_c]
        sg = [sg0, sg1, sg2]
        si = [si0, si1, si2]

        pltpu.sync_copy(src_hbm.at[wid, 0], srcr.at[0])
        pltpu.sync_copy(dst_hbm.at[wid, 0], dstr.at[0])
        pltpu.sync_copy(zeros_hbm, acc_s.at[pl.ds(sid * rt, rt)])
        pltpu.async_copy(xs_hbm.at[srcr.at[0]], rows_a, sg0)
        pltpu.async_copy(src_hbm.at[wid, 1], srcr.at[1], si1)
        pltpu.async_copy(dst_hbm.at[wid, 1], dstr.at[1], si1)
        plsc.subcore_barrier()

        def body(t, carry):
            i0 = 3 * t
            for k_ in range(3):
                i = i0 + k_
                s0, s1, s2 = k_, (k_ + 1) % 3, (k_ + 2) % 3
                pltpu.make_async_copy(
                    xs_hbm.at[srcr.at[s0]], rows[s0], sg[s0]).wait()
                pltpu.make_async_copy(
                    src_hbm.at[wid, i + 1], srcr.at[s1], si[s1]).wait()
                pltpu.make_async_copy(
                    dst_hbm.at[wid, i + 1], dstr.at[s1], si[s1]).wait()
                pltpu.async_copy(xs_hbm.at[srcr.at[s1]], rows[s1], sg[s1])
                pltpu.async_copy(src_hbm.at[wid, i + 2], srcr.at[s2], si[s2])
                pltpu.async_copy(dst_hbm.at[wid, i + 2], dstr.at[s2], si[s2])
                pltpu.sync_copy(rows[s0], acc_s.at[dstr.at[s0]], add=True)
            return carry

        lax.fori_loop(0, t_steps, body, 0)

        z = iters % 3
        z1 = (iters + 1) % 3
        pltpu.make_async_copy(xs_hbm.at[srcr.at[z]], rows[z], sg[z]).wait()
        pltpu.make_async_copy(
            src_hbm.at[wid, iters + 1], srcr.at[z1], si[z1]).wait()
        pltpu.make_async_copy(
            dst_hbm.at[wid, iters + 1], dstr.at[z1], si[z1]).wait()

        plsc.subcore_barrier()
        pltpu.sync_copy(acc_s.at[pl.ds(sid * rt, rt)],
                        out_hbm.at[cid, pl.ds(sid * rt, rt)])

    return k


def _dinv_from(degp_ref):
    deg = degp_ref[0, :, 0] + degp_ref[1, :, 0] + 1.0
    return lax.rsqrt(jnp.maximum(deg, 1e-12))


def _xs_body(x_ref, w_ref, degp_ref, o_ref):
    dinv = _dinv_from(degp_ref)
    xw = jnp.dot(x_ref[...], w_ref[...], preferred_element_type=jnp.float32)
    o_ref[...] = xw * dinv[:, None]


def _h_stats_body(nblk, n, aggp_ref, xs_ref, degp_ref, b1_ref,
                  h_ref, stats_ref, acc_ref):
    i = pl.program_id(0)
    dinv = _dinv_from(degp_ref)
    h = (aggp_ref[0] + aggp_ref[1] + xs_ref[...]) * dinv[:, None] + b1_ref[...]
    h_ref[...] = h

    @pl.when(i == 0)
    def _():
        acc_ref[...] = jnp.zeros_like(acc_ref)

    acc_ref[0:1] += jnp.sum(h, axis=0, keepdims=True)
    acc_ref[1:2] += jnp.sum(h * h, axis=0, keepdims=True)

    @pl.when(i == nblk - 1)
    def _():
        mean = acc_ref[0:1] / n
        var = acc_ref[1:2] / n - mean * mean
        stats_ref[0:1] = mean
        stats_ref[1:2] = lax.rsqrt(var + 1e-5)


def _mlp_body(h_ref, stats_ref, g_ref, be_ref, a1_ref, wp1_ref, bp1_ref,
              ap_ref, wp2_ref, bp2_ref, o_ref):
    hn = (h_ref[...] - stats_ref[0:1]) * stats_ref[1:2] * g_ref[...] + be_ref[...]
    a1 = a1_ref[0, 0]
    p = jnp.where(hn >= 0, hn, a1 * hn)
    q1 = jnp.dot(p, wp1_ref[...], preferred_element_type=jnp.float32) + bp1_ref[...]
    ap = ap_ref[0, 0]
    q1 = jnp.where(q1 >= 0, q1, ap * q1)
    o_ref[...] = jnp.dot(q1, wp2_ref[...], preferred_element_type=jnp.float32) + bp2_ref[...]


def kernel(x, edge_index, W1, b1, gamma1, beta1, a1, Wp1, bp1, ap, Wp2, bp2):
    n, d = x.shape
    e = edge_index.shape[1]
    d_pred = Wp1.shape[1]

    n_pad = -(-(n + 1) // (NS * 8)) * (NS * 8)
    rt = n_pad // NS
    iters = -(-e // (NW * EB))
    iters = -(-iters // 3) * 3
    iters_pad = -(-(iters + 2) // 8) * 8
    e_pad = iters * NW * EB

    src = jnp.pad(
        jnp.concatenate([edge_index[0], jnp.zeros((e_pad - e,), jnp.int32)]
                        ).reshape(NW, iters, EB),
        ((0, 0), (0, iters_pad - iters), (0, 0)))
    dst = jnp.pad(
        jnp.concatenate([edge_index[1], jnp.full((e_pad - e,), n, jnp.int32)]
                        ).reshape(NW, iters, EB),
        ((0, 0), (0, iters_pad - iters), (0, 0)),
        constant_values=n)

    ones_rows = jnp.ones((EB, d), jnp.float32)
    zeros_rows = jnp.zeros((rt, d), jnp.float32)

    degp = _deg_kernel(n_pad, iters, iters_pad, d)(dst, ones_rows, zeros_rows)

    nblk = 5 if n % 5 == 0 else 1
    bn = n // nblk
    grid = (nblk,)

    xs = pl.pallas_call(
        _xs_body,
        grid=grid,
        in_specs=[
            pl.BlockSpec((bn, d), lambda i: (i, 0)),
            pl.BlockSpec((d, d), lambda i: (0, 0)),
            pl.BlockSpec((NC, bn, d), lambda i: (0, i, 0)),
        ],
        out_specs=pl.BlockSpec((bn, d), lambda i: (i, 0)),
        out_shape=jax.ShapeDtypeStruct((n, d), jnp.float32),
    )(x, W1, degp)

    aggp = _agg_kernel(n_pad, iters, iters_pad, d)(xs, src, dst, zeros_rows)

    h, stats = pl.pallas_call(
        functools.partial(_h_stats_body, nblk, float(n)),
        grid=grid,
        in_specs=[
            pl.BlockSpec((NC, bn, d), lambda i: (0, i, 0)),
            pl.BlockSpec((bn, d), lambda i: (i, 0)),
            pl.BlockSpec((NC, bn, d), lambda i: (0, i, 0)),
            pl.BlockSpec((1, d), lambda i: (0, 0)),
        ],
        out_specs=[
            pl.BlockSpec((bn, d), lambda i: (i, 0)),
            pl.BlockSpec((2, d), lambda i: (0, 0)),
        ],
        out_shape=[
            jax.ShapeDtypeStruct((n, d), jnp.float32),
            jax.ShapeDtypeStruct((2, d), jnp.float32),
        ],
        scratch_shapes=[pltpu.VMEM((2, d), jnp.float32)],
    )(aggp, xs, degp, b1.reshape(1, d))

    q = pl.pallas_call(
        _mlp_body,
        grid=grid,
        in_specs=[
            pl.BlockSpec((bn, d), lambda i: (i, 0)),
            pl.BlockSpec((2, d), lambda i: (0, 0)),
            pl.BlockSpec((1, d), lambda i: (0, 0)),
            pl.BlockSpec((1, d), lambda i: (0, 0)),
            pl.BlockSpec((1, 1), lambda i: (0, 0)),
            pl.BlockSpec((d, d_pred), lambda i: (0, 0)),
            pl.BlockSpec((1, d_pred), lambda i: (0, 0)),
            pl.BlockSpec((1, 1), lambda i: (0, 0)),
            pl.BlockSpec((d_pred, d), lambda i: (0, 0)),
            pl.BlockSpec((1, d), lambda i: (0, 0)),
        ],
        out_specs=pl.BlockSpec((bn, d), lambda i: (i, 0)),
        out_shape=jax.ShapeDtypeStruct((n, d), jnp.float32),
    )(h, stats, gamma1.reshape(1, d), beta1.reshape(1, d),
      a1.reshape(1, 1), Wp1, bp1.reshape(1, d_pred), ap.reshape(1, 1),
      Wp2, bp2.reshape(1, d))

    return q


# speedup vs baseline: 1.2047x; 1.2047x over previous
import functools

import jax
import jax.numpy as jnp
from jax import lax
from jax.experimental import pallas as pl
from jax.experimental.pallas import tpu as pltpu
from jax.experimental.pallas import tpu_sc as plsc

NC = 2
NS = 16
NW = NC * NS
EB = 128


def _sc_mesh():
    return plsc.VectorSubcoreMesh(core_axis_name="c", subcore_axis_name="s")


def _deg_kernel(n_pad, iters, iters_pad, d):
    rt = n_pad // NS
    fire = 8
    rounds = iters // fire
    tail = iters - rounds * fire

    @functools.partial(
        pl.kernel,
        out_type=jax.ShapeDtypeStruct((NC, n_pad, d), jnp.float32),
        mesh=_sc_mesh(),
        scratch_types=[
            pltpu.VMEM((iters_pad, EB), jnp.int32),
            pltpu.VMEM((EB, d), jnp.float32),
            pltpu.SemaphoreType.DMA,
            pltpu.VMEM_SHARED((n_pad, d), jnp.float32),
        ],
    )
    def k(dst_hbm, ones_hbm, zeros_hbm, out_hbm, idx_v, ones_v, sem, acc_s):
        cid = lax.axis_index("c")
        sid = lax.axis_index("s")
        wid = sid * NC + cid
        pltpu.sync_copy(dst_hbm.at[wid], idx_v)
        pltpu.sync_copy(ones_hbm, ones_v)
        pltpu.sync_copy(zeros_hbm, acc_s.at[pl.ds(sid * rt, rt)])
        plsc.subcore_barrier()

        def body(r, carry):
            for j in range(fire):
                pltpu.async_copy(ones_v, acc_s.at[idx_v.at[r * fire + j]],
                                 sem, add=True)
            for j in range(fire):
                pltpu.make_async_copy(
                    ones_v, acc_s.at[idx_v.at[r * fire + j]], sem).wait()
            return carry

        lax.fori_loop(0, rounds, body, 0)
        for j in range(tail):
            pltpu.async_copy(ones_v, acc_s.at[idx_v.at[rounds * fire + j]],
                             sem, add=True)
        for j in range(tail):
            pltpu.make_async_copy(
                ones_v, acc_s.at[idx_v.at[rounds * fire + j]], sem).wait()
        plsc.subcore_barrier()
        pltpu.sync_copy(acc_s.at[pl.ds(sid * rt, rt)],
                        out_hbm.at[cid, pl.ds(sid * rt, rt)])

    return k


def _agg_kernel(n_pad, iters, iters_pad, d):
    rt = n_pad // NS
    C = 8
    chunks = iters // C
    P = chunks // 2

    @functools.partial(
        pl.kernel,
        out_type=jax.ShapeDtypeStruct((NC, n_pad, d), jnp.float32),
        mesh=_sc_mesh(),
        scratch_types=[
            pltpu.VMEM((C, EB), jnp.int32),
            pltpu.VMEM((C, EB), jnp.int32),
            pltpu.VMEM((iters, EB), jnp.int32),
            pltpu.VMEM((EB, d), jnp.float32),
            pltpu.VMEM((EB, d), jnp.float32),
            pltpu.VMEM_SHARED((n_pad, d), jnp.float32),
            pltpu.SemaphoreType.DMA,
            pltpu.SemaphoreType.DMA,
            pltpu.SemaphoreType.DMA,
            pltpu.SemaphoreType.DMA,
            pltpu.SemaphoreType.DMA,
            pltpu.SemaphoreType.DMA,
        ],
    )
    def k(xs_hbm, src_hbm, dst_hbm, zeros_hbm, out_hbm,
          src0, src1, dst_v, rows_a, rows_b, acc_s,
          sg0, sg1, ss0, ss1, sc0, sc1):
        cid = lax.axis_index("c")
        sid = lax.axis_index("s")
        wid = sid * NC + cid
        srcb = [src0, src1]
        rows = [rows_a, rows_b]
        sg = [sg0, sg1]
        ss = [ss0, ss1]
        sc = [sc0, sc1]

        pltpu.sync_copy(src_hbm.at[wid, pl.ds(0, C)], src0)
        pltpu.async_copy(src_hbm.at[wid, pl.ds(C, C)], src1, sc1)
        pltpu.sync_copy(dst_hbm.at[wid, pl.ds(0, iters)], dst_v)
        pltpu.sync_copy(zeros_hbm.at[pl.ds(0, EB)], rows_b)
        pltpu.sync_copy(zeros_hbm, acc_s.at[pl.ds(sid * rt, rt)])
        plsc.subcore_barrier()
        pltpu.async_copy(xs_hbm.at[src0.at[0]], rows_a, sg0)
        pltpu.async_copy(rows_b, acc_s.at[dst_v.at[0]], ss1, add=True)

        def body(p, carry):
            for k_ in range(2 * C):
                i = 2 * C * p + k_
                c_cur = 2 * p + k_ // C
                s = k_ % 2
                r = k_ % C
                b = (k_ // C) % 2
                r1 = (k_ + 1) % C
                b1 = ((k_ + 1) // C) % 2
                pltpu.make_async_copy(
                    xs_hbm.at[srcb[b].at[r]], rows[s], sg[s]).wait()
                if r == C - 1:
                    pltpu.async_copy(
                        src_hbm.at[wid, pl.ds(C * (c_cur + 2), C)],
                        srcb[b], sc[b])
                    pltpu.make_async_copy(
                        src_hbm.at[wid, pl.ds(C * (c_cur + 1), C)],
                        srcb[1 - b], sc[1 - b]).wait()
                pltpu.make_async_copy(
                    rows[1 - s], acc_s.at[dst_v.at[0]], ss[1 - s]).wait()
                pltpu.async_copy(
                    xs_hbm.at[srcb[b1].at[r1]], rows[1 - s], sg[1 - s])
                pltpu.async_copy(
                    rows[s], acc_s.at[dst_v.at[i]], ss[s], add=True)
            return carry

        lax.fori_loop(0, P, body, 0)

        pltpu.make_async_copy(
            rows[1], acc_s.at[dst_v.at[0]], ss[1]).wait()
        pltpu.make_async_copy(xs_hbm.at[src0.at[0]], rows[0], sg[0]).wait()
        pltpu.make_async_copy(
            src_hbm.at[wid, pl.ds(C * (chunks + 1), C)], src1, sc1).wait()

        plsc.subcore_barrier()
        pltpu.sync_copy(acc_s.at[pl.ds(sid * rt, rt)],
                        out_hbm.at[cid, pl.ds(sid * rt, rt)])

    return k


def _dinv_from(degp_ref):
    deg = degp_ref[0, :, 0] + degp_ref[1, :, 0] + 1.0
    return lax.rsqrt(jnp.maximum(deg, 1e-12))


def _xs_body(x_ref, w_ref, degp_ref, o_ref):
    dinv = _dinv_from(degp_ref)
    xw = jnp.dot(x_ref[...], w_ref[...], preferred_element_type=jnp.float32)
    o_ref[...] = xw * dinv[:, None]


def _h_stats_body(nblk, n, aggp_ref, xs_ref, degp_ref, b1_ref,
                  h_ref, stats_ref, acc_ref):
    i = pl.program_id(0)
    dinv = _dinv_from(degp_ref)
    h = (aggp_ref[0] + aggp_ref[1] + xs_ref[...]) * dinv[:, None] + b1_ref[...]
    h_ref[...] = h

    @pl.when(i == 0)
    def _():
        acc_ref[...] = jnp.zeros_like(acc_ref)

    acc_ref[0:1] += jnp.sum(h, axis=0, keepdims=True)
    acc_ref[1:2] += jnp.sum(h * h, axis=0, keepdims=True)

    @pl.when(i == nblk - 1)
    def _():
        mean = acc_ref[0:1] / n
        var = acc_ref[1:2] / n - mean * mean
        stats_ref[0:1] = mean
        stats_ref[1:2] = lax.rsqrt(var + 1e-5)


def _mlp_body(h_ref, stats_ref, g_ref, be_ref, a1_ref, wp1_ref, bp1_ref,
              ap_ref, wp2_ref, bp2_ref, o_ref):
    hn = (h_ref[...] - stats_ref[0:1]) * stats_ref[1:2] * g_ref[...] + be_ref[...]
    a1 = a1_ref[0, 0]
    p = jnp.where(hn >= 0, hn, a1 * hn)
    q1 = jnp.dot(p, wp1_ref[...], preferred_element_type=jnp.float32) + bp1_ref[...]
    ap = ap_ref[0, 0]
    q1 = jnp.where(q1 >= 0, q1, ap * q1)
    o_ref[...] = jnp.dot(q1, wp2_ref[...], preferred_element_type=jnp.float32) + bp2_ref[...]


def kernel(x, edge_index, W1, b1, gamma1, beta1, a1, Wp1, bp1, ap, Wp2, bp2):
    n, d = x.shape
    e = edge_index.shape[1]
    d_pred = Wp1.shape[1]

    n_pad = -(-(n + 1) // (NS * 8)) * (NS * 8)
    rt = n_pad // NS
    iters = -(-e // (NW * EB))
    iters = -(-iters // 16) * 16
    iters_pad = iters + 16
    e_pad = iters * NW * EB

    src = jnp.pad(
        jnp.concatenate([edge_index[0], jnp.zeros((e_pad - e,), jnp.int32)]
                        ).reshape(NW, iters, EB),
        ((0, 0), (0, iters_pad - iters), (0, 0)))
    dst = jnp.pad(
        jnp.concatenate([edge_index[1], jnp.full((e_pad - e,), n, jnp.int32)]
                        ).reshape(NW, iters, EB),
        ((0, 0), (0, iters_pad - iters), (0, 0)),
        constant_values=n)

    ones_rows = jnp.ones((EB, d), jnp.float32)
    zeros_rows = jnp.zeros((rt, d), jnp.float32)

    degp = _deg_kernel(n_pad, iters, iters_pad, d)(dst, ones_rows, zeros_rows)

    nblk = 5 if n % 5 == 0 else 1
    bn = n // nblk
    grid = (nblk,)

    xs = pl.pallas_call(
        _xs_body,
        grid=grid,
        in_specs=[
            pl.BlockSpec((bn, d), lambda i: (i, 0)),
            pl.BlockSpec((d, d), lambda i: (0, 0)),
            pl.BlockSpec((NC, bn, d), lambda i: (0, i, 0)),
        ],
        out_specs=pl.BlockSpec((bn, d), lambda i: (i, 0)),
        out_shape=jax.ShapeDtypeStruct((n, d), jnp.float32),
    )(x, W1, degp)

    aggp = _agg_kernel(n_pad, iters, iters_pad, d)(xs, src, dst, zeros_rows)

    h, stats = pl.pallas_call(
        functools.partial(_h_stats_body, nblk, float(n)),
        grid=grid,
        in_specs=[
            pl.BlockSpec((NC, bn, d), lambda i: (0, i, 0)),
            pl.BlockSpec((bn, d), lambda i: (i, 0)),
            pl.BlockSpec((NC, bn, d), lambda i: (0, i, 0)),
            pl.BlockSpec((1, d), lambda i: (0, 0)),
        ],
        out_specs=[
            pl.BlockSpec((bn, d), lambda i: (i, 0)),
            pl.BlockSpec((2, d), lambda i: (0, 0)),
        ],
        out_shape=[
            jax.ShapeDtypeStruct((n, d), jnp.float32),
            jax.ShapeDtypeStruct((2, d), jnp.float32),
        ],
        scratch_shapes=[pltpu.VMEM((2, d), jnp.float32)],
    )(aggp, xs, degp, b1.reshape(1, d))

    q = pl.pallas_call(
        _mlp_body,
        grid=grid,
        in_specs=[
            pl.BlockSpec((bn, d), lambda i: (i, 0)),
            pl.BlockSpec((2, d), lambda i: (0, 0)),
            pl.BlockSpec((1, d), lambda i: (0, 0)),
            pl.BlockSpec((1, d), lambda i: (0, 0)),
            pl.BlockSpec((1, 1), lambda i: (0, 0)),
            pl.BlockSpec((d, d_pred), lambda i: (0, 0)),
            pl.BlockSpec((1, d_pred), lambda i: (0, 0)),
            pl.BlockSpec((1, 1), lambda i: (0, 0)),
            pl.BlockSpec((d_pred, d), lambda i: (0, 0)),
            pl.BlockSpec((1, d), lambda i: (0, 0)),
        ],
        out_specs=pl.BlockSpec((bn, d), lambda i: (i, 0)),
        out_shape=jax.ShapeDtypeStruct((n, d), jnp.float32),
    )(h, stats, gamma1.reshape(1, d), beta1.reshape(1, d),
      a1.reshape(1, 1), Wp1, bp1.reshape(1, d_pred), ap.reshape(1, 1),
      Wp2, bp2.reshape(1, d))

    return q


# speedup vs baseline: 2.3942x; 1.9873x over previous
import functools

import jax
import jax.numpy as jnp
from jax import lax
from jax.experimental import pallas as pl
from jax.experimental.pallas import tpu as pltpu
from jax.experimental.pallas import tpu_sc as plsc

NC = 2
NS = 16
NW = NC * NS
EB = 128


def _sc_mesh():
    return plsc.VectorSubcoreMesh(core_axis_name="c", subcore_axis_name="s")


def _deg_kernel(n_pad, iters, iters_pad, d):
    rt = n_pad // NS
    fire = 8
    rounds = iters // fire
    tail = iters - rounds * fire

    @functools.partial(
        pl.kernel,
        out_type=jax.ShapeDtypeStruct((NC, n_pad, d), jnp.float32),
        mesh=_sc_mesh(),
        scratch_types=[
            pltpu.VMEM((iters_pad, EB), jnp.int32),
            pltpu.VMEM((EB, d), jnp.float32),
            pltpu.SemaphoreType.DMA,
            pltpu.VMEM_SHARED((n_pad, d), jnp.float32),
        ],
    )
    def k(dst_hbm, ones_hbm, zeros_hbm, out_hbm, idx_v, ones_v, sem, acc_s):
        cid = lax.axis_index("c")
        sid = lax.axis_index("s")
        wid = sid * NC + cid
        pltpu.sync_copy(dst_hbm.at[wid], idx_v)
        pltpu.sync_copy(ones_hbm, ones_v)
        pltpu.sync_copy(zeros_hbm, acc_s.at[pl.ds(sid * rt, rt)])
        plsc.subcore_barrier()

        def body(r, carry):
            for j in range(fire):
                pltpu.async_copy(ones_v, acc_s.at[idx_v.at[r * fire + j]],
                                 sem, add=True)
            for j in range(fire):
                pltpu.make_async_copy(
                    ones_v, acc_s.at[idx_v.at[r * fire + j]], sem).wait()
            return carry

        lax.fori_loop(0, rounds, body, 0)
        for j in range(tail):
            pltpu.async_copy(ones_v, acc_s.at[idx_v.at[rounds * fire + j]],
                             sem, add=True)
        for j in range(tail):
            pltpu.make_async_copy(
                ones_v, acc_s.at[idx_v.at[rounds * fire + j]], sem).wait()
        plsc.subcore_barrier()
        pltpu.sync_copy(acc_s.at[pl.ds(sid * rt, rt)],
                        out_hbm.at[cid, pl.ds(sid * rt, rt)])

    return k


def _agg_kernel(n_pad, iters0, iters1, iters_pad, d):
    rt = n_pad // NS

    @functools.partial(
        pl.kernel,
        out_type=jax.ShapeDtypeStruct((NC, n_pad, d), jnp.float32),
        mesh=_sc_mesh(),
        scratch_types=[
            pltpu.VMEM((EB,), jnp.int32),
            pltpu.VMEM((EB,), jnp.int32),
            pltpu.VMEM((iters_pad, EB), jnp.int32),
            pltpu.VMEM((EB, d), jnp.float32),
            pltpu.VMEM((EB, d), jnp.float32),
            pltpu.VMEM_SHARED((n_pad, d), jnp.float32),
            pltpu.SemaphoreType.DMA,
            pltpu.SemaphoreType.DMA,
            pltpu.SemaphoreType.DMA,
            pltpu.SemaphoreType.DMA,
        ],
    )
    def k(xs_hbm, src_hbm, dst_hbm, zeros_hbm, out_hbm,
          src_a, src_b, dst_v, rows_a, rows_b, acc_s,
          sem_a, sem_b, sem_ia, sem_ib):
        cid = lax.axis_index("c")
        sid = lax.axis_index("s")
        wid = sid * NC + cid
        iters_w = jnp.where(cid == 0, iters0, iters1)
        pairs_w = (iters_w - 1) // 2
        pltpu.sync_copy(dst_hbm.at[wid], dst_v)
        pltpu.sync_copy(src_hbm.at[wid, 0], src_a)
        pltpu.sync_copy(zeros_hbm, acc_s.at[pl.ds(sid * rt, rt)])
        pltpu.async_copy(xs_hbm.at[src_a], rows_a, sem_a)
        pltpu.async_copy(src_hbm.at[wid, 1], src_b, sem_ib)
        plsc.subcore_barrier()

        def body(g, carry):
            i0 = 2 * g
            pltpu.make_async_copy(xs_hbm.at[src_a], rows_a, sem_a).wait()
            pltpu.async_copy(src_hbm.at[wid, i0 + 2], src_a, sem_ia)
            pltpu.make_async_copy(src_hbm.at[wid, i0 + 1], src_b, sem_ib).wait()
            pltpu.async_copy(xs_hbm.at[src_b], rows_b, sem_b)
            pltpu.sync_copy(rows_a, acc_s.at[dst_v.at[i0]], add=True)
            pltpu.make_async_copy(xs_hbm.at[src_b], rows_b, sem_b).wait()
            pltpu.async_copy(src_hbm.at[wid, i0 + 3], src_b, sem_ib)
            pltpu.make_async_copy(src_hbm.at[wid, i0 + 2], src_a, sem_ia).wait()
            pltpu.async_copy(xs_hbm.at[src_a], rows_a, sem_a)
            pltpu.sync_copy(rows_b, acc_s.at[dst_v.at[i0 + 1]], add=True)
            return carry

        lax.fori_loop(0, pairs_w, body, 0)
        last = iters_w - 1
        pltpu.make_async_copy(xs_hbm.at[src_a], rows_a, sem_a).wait()
        pltpu.sync_copy(rows_a, acc_s.at[dst_v.at[last]], add=True)
        pltpu.make_async_copy(src_hbm.at[wid, 0], src_b, sem_ib).wait()

        plsc.subcore_barrier()
        pltpu.sync_copy(acc_s.at[pl.ds(sid * rt, rt)],
                        out_hbm.at[cid, pl.ds(sid * rt, rt)])

    return k


def _dinv_from(degp_ref):
    deg = degp_ref[0, :, 0] + degp_ref[1, :, 0] + 1.0
    return lax.rsqrt(jnp.maximum(deg, 1e-12))


def _xs_body(x_ref, w_ref, degp_ref, o_ref):
    dinv = _dinv_from(degp_ref)
    xw = jnp.dot(x_ref[...], w_ref[...], preferred_element_type=jnp.float32)
    o_ref[...] = xw * dinv[:, None]


def _h_stats_body(nblk, n, aggp_ref, xs_ref, degp_ref, b1_ref,
                  h_ref, stats_ref, acc_ref):
    i = pl.program_id(0)
    dinv = _dinv_from(degp_ref)
    h = (aggp_ref[0] + aggp_ref[1] + xs_ref[...]) * dinv[:, None] + b1_ref[...]
    h_ref[...] = h

    @pl.when(i == 0)
    def _():
        acc_ref[...] = jnp.zeros_like(acc_ref)

    acc_ref[0:1] += jnp.sum(h, axis=0, keepdims=True)
    acc_ref[1:2] += jnp.sum(h * h, axis=0, keepdims=True)

    @pl.when(i == nblk - 1)
    def _():
        mean = acc_ref[0:1] / n
        var = acc_ref[1:2] / n - mean * mean
        stats_ref[0:1] = mean
        stats_ref[1:2] = lax.rsqrt(var + 1e-5)


def _mlp_body(h_ref, stats_ref, g_ref, be_ref, a1_ref, wp1_ref, bp1_ref,
              ap_ref, wp2_ref, bp2_ref, o_ref):
    hn = (h_ref[...] - stats_ref[0:1]) * stats_ref[1:2] * g_ref[...] + be_ref[...]
    a1 = a1_ref[0, 0]
    p = jnp.where(hn >= 0, hn, a1 * hn)
    q1 = jnp.dot(p, wp1_ref[...], preferred_element_type=jnp.float32) + bp1_ref[...]
    ap = ap_ref[0, 0]
    q1 = jnp.where(q1 >= 0, q1, ap * q1)
    o_ref[...] = jnp.dot(q1, wp2_ref[...], preferred_element_type=jnp.float32) + bp2_ref[...]


def kernel(x, edge_index, W1, b1, gamma1, beta1, a1, Wp1, bp1, ap, Wp2, bp2):
    n, d = x.shape
    e = edge_index.shape[1]
    d_pred = Wp1.shape[1]

    n_pad = -(-(n + 1) // (NS * 16)) * (NS * 16)
    rt = n_pad // NS
    nb = -(-e // EB)

    iters_d = -(-nb // NW)
    if iters_d % 2 == 0:
        iters_d += 1
    itp_d = -(-iters_d // 8) * 8
    ed_pad = iters_d * NW * EB
    dst_deg = jnp.pad(
        jnp.concatenate([edge_index[1], jnp.full((ed_pad - e,), n, jnp.int32)]
                        ).reshape(NW, iters_d, EB),
        ((0, 0), (0, itp_d - iters_d), (0, 0)), constant_values=n)

    S = -(-nb // NS)
    if S % 2 == 1:
        S += 1
    iters0 = int(0.7 * S) | 1
    iters1 = S - iters0
    itp_a = -(-(max(iters0, iters1) + 1) // 8) * 8
    ea_pad = S * NS * EB

    def plane(v, fill):
        flat = jnp.concatenate(
            [v, jnp.full((ea_pad - e,), fill, jnp.int32)])
        c0 = flat[:NS * iters0 * EB].reshape(NS, iters0, EB)
        c1 = flat[NS * iters0 * EB:].reshape(NS, iters1, EB)
        c0 = jnp.pad(c0, ((0, 0), (0, itp_a - iters0), (0, 0)),
                     constant_values=fill)
        c1 = jnp.pad(c1, ((0, 0), (0, itp_a - iters1), (0, 0)),
                     constant_values=fill)
        return jnp.stack([c0, c1], axis=1).reshape(NW, itp_a, EB)

    src_agg = plane(edge_index[0], 0)
    dst_agg = plane(edge_index[1], n)

    ones_rows = jnp.ones((EB, d), jnp.float32)
    zeros_rows = jnp.zeros((rt, d), jnp.float32)

    degp = _deg_kernel(n_pad, iters_d, itp_d, d)(dst_deg, ones_rows, zeros_rows)

    nblk = 5 if n % 5 == 0 else 1
    bn = n // nblk
    grid = (nblk,)

    xs = pl.pallas_call(
        _xs_body,
        grid=grid,
        in_specs=[
            pl.BlockSpec((bn, d), lambda i: (i, 0)),
            pl.BlockSpec((d, d), lambda i: (0, 0)),
            pl.BlockSpec((NC, bn, d), lambda i: (0, i, 0)),
        ],
        out_specs=pl.BlockSpec((bn, d), lambda i: (i, 0)),
        out_shape=jax.ShapeDtypeStruct((n, d), jnp.float32),
    )(x, W1, degp)

    aggp = _agg_kernel(n_pad, iters0, iters1, itp_a, d)(xs, src_agg, dst_agg, zeros_rows)

    h, stats = pl.pallas_call(
        functools.partial(_h_stats_body, nblk, float(n)),
        grid=grid,
        in_specs=[
            pl.BlockSpec((NC, bn, d), lambda i: (0, i, 0)),
            pl.BlockSpec((bn, d), lambda i: (i, 0)),
            pl.BlockSpec((NC, bn, d), lambda i: (0, i, 0)),
            pl.BlockSpec((1, d), lambda i: (0, 0)),
        ],
        out_specs=[
            pl.BlockSpec((bn, d), lambda i: (i, 0)),
            pl.BlockSpec((2, d), lambda i: (0, 0)),
        ],
        out_shape=[
            jax.ShapeDtypeStruct((n, d), jnp.float32),
            jax.ShapeDtypeStruct((2, d), jnp.float32),
        ],
        scratch_shapes=[pltpu.VMEM((2, d), jnp.float32)],
    )(aggp, xs, degp, b1.reshape(1, d))

    q = pl.pallas_call(
        _mlp_body,
        grid=grid,
        in_specs=[
            pl.BlockSpec((bn, d), lambda i: (i, 0)),
            pl.BlockSpec((2, d), lambda i: (0, 0)),
            pl.BlockSpec((1, d), lambda i: (0, 0)),
            pl.BlockSpec((1, d), lambda i: (0, 0)),
            pl.BlockSpec((1, 1), lambda i: (0, 0)),
            pl.BlockSpec((d, d_pred), lambda i: (0, 0)),
            pl.BlockSpec((1, d_pred), lambda i: (0, 0)),
            pl.BlockSpec((1, 1), lambda i: (0, 0)),
            pl.BlockSpec((d_pred, d), lambda i: (0, 0)),
            pl.BlockSpec((1, d), lambda i: (0, 0)),
        ],
        out_specs=pl.BlockSpec((bn, d), lambda i: (i, 0)),
        out_shape=jax.ShapeDtypeStruct((n, d), jnp.float32),
    )(h, stats, gamma1.reshape(1, d), beta1.reshape(1, d),
      a1.reshape(1, 1), Wp1, bp1.reshape(1, d_pred), ap.reshape(1, 1),
      Wp2, bp2.reshape(1, d))

    return q


# speedup vs baseline: 2.4675x; 1.0306x over previous
import functools

import jax
import jax.numpy as jnp
from jax import lax
from jax.experimental import pallas as pl
from jax.experimental.pallas import tpu as pltpu
from jax.experimental.pallas import tpu_sc as plsc

NC = 2
NS = 16
NW = NC * NS
EB = 128


def _sc_mesh():
    return plsc.VectorSubcoreMesh(core_axis_name="c", subcore_axis_name="s")


def _deg_kernel(n_pad, iters, iters_pad, d):
    rt = n_pad // NS
    fire = 8
    rounds = iters // fire
    tail = iters - rounds * fire

    @functools.partial(
        pl.kernel,
        out_type=jax.ShapeDtypeStruct((NC, n_pad, d), jnp.float32),
        mesh=_sc_mesh(),
        scratch_types=[
            pltpu.VMEM((iters_pad, EB), jnp.int32),
            pltpu.VMEM((EB, d), jnp.float32),
            pltpu.SemaphoreType.DMA,
            pltpu.VMEM_SHARED((n_pad, d), jnp.float32),
        ],
    )
    def k(dst_hbm, ones_hbm, zeros_hbm, out_hbm, idx_v, ones_v, sem, acc_s):
        cid = lax.axis_index("c")
        sid = lax.axis_index("s")
        wid = sid * NC + cid
        pltpu.sync_copy(dst_hbm.at[wid], idx_v)
        pltpu.sync_copy(ones_hbm, ones_v)
        pltpu.sync_copy(zeros_hbm, acc_s.at[pl.ds(sid * rt, rt)])
        plsc.subcore_barrier()

        def body(r, carry):
            for j in range(fire):
                pltpu.async_copy(ones_v, acc_s.at[idx_v.at[r * fire + j]],
                                 sem, add=True)
            for j in range(fire):
                pltpu.make_async_copy(
                    ones_v, acc_s.at[idx_v.at[r * fire + j]], sem).wait()
            return carry

        lax.fori_loop(0, rounds, body, 0)
        for j in range(tail):
            pltpu.async_copy(ones_v, acc_s.at[idx_v.at[rounds * fire + j]],
                             sem, add=True)
        for j in range(tail):
            pltpu.make_async_copy(
                ones_v, acc_s.at[idx_v.at[rounds * fire + j]], sem).wait()
        plsc.subcore_barrier()
        pltpu.sync_copy(acc_s.at[pl.ds(sid * rt, rt)],
                        out_hbm.at[cid, pl.ds(sid * rt, rt)])

    return k


def _agg_kernel(n_pad, iters0, iters1, iters_pad, d):
    rt = n_pad // NS

    @functools.partial(
        pl.kernel,
        out_type=jax.ShapeDtypeStruct((NC, n_pad, d), jnp.float32),
        mesh=_sc_mesh(),
        scratch_types=[
            pltpu.VMEM((EB,), jnp.int32),
            pltpu.VMEM((EB,), jnp.int32),
            pltpu.VMEM((iters_pad, EB), jnp.int32),
            pltpu.VMEM((EB, d), jnp.float32),
            pltpu.VMEM((EB, d), jnp.float32),
            pltpu.VMEM_SHARED((n_pad, d), jnp.float32),
            pltpu.SemaphoreType.DMA,
            pltpu.SemaphoreType.DMA,
            pltpu.SemaphoreType.DMA,
            pltpu.SemaphoreType.DMA,
        ],
    )
    def k(xs_hbm, src_hbm, dst_hbm, zeros_hbm, out_hbm,
          src_a, src_b, dst_v, rows_a, rows_b, acc_s,
          sem_a, sem_b, sem_ia, sem_ib):
        cid = lax.axis_index("c")
        sid = lax.axis_index("s")
        wid = sid * NC + cid
        iters_w = jnp.where(cid == 0, iters0, iters1)
        pairs_w = (iters_w - 1) // 2
        pltpu.sync_copy(dst_hbm.at[wid], dst_v)
        pltpu.sync_copy(src_hbm.at[wid, 0], src_a)
        pltpu.sync_copy(zeros_hbm, acc_s.at[pl.ds(sid * rt, rt)])
        pltpu.async_copy(xs_hbm.at[src_a], rows_a, sem_a)
        pltpu.async_copy(src_hbm.at[wid, 1], src_b, sem_ib)
        plsc.subcore_barrier()

        def body(g, carry):
            i0 = 2 * g
            pltpu.make_async_copy(xs_hbm.at[src_a], rows_a, sem_a).wait()
            pltpu.async_copy(src_hbm.at[wid, i0 + 2], src_a, sem_ia)
            pltpu.make_async_copy(src_hbm.at[wid, i0 + 1], src_b, sem_ib).wait()
            pltpu.async_copy(xs_hbm.at[src_b], rows_b, sem_b)
            pltpu.sync_copy(rows_a, acc_s.at[dst_v.at[i0]], add=True)
            pltpu.make_async_copy(xs_hbm.at[src_b], rows_b, sem_b).wait()
            pltpu.async_copy(src_hbm.at[wid, i0 + 3], src_b, sem_ib)
            pltpu.make_async_copy(src_hbm.at[wid, i0 + 2], src_a, sem_ia).wait()
            pltpu.async_copy(xs_hbm.at[src_a], rows_a, sem_a)
            pltpu.sync_copy(rows_b, acc_s.at[dst_v.at[i0 + 1]], add=True)
            return carry

        lax.fori_loop(0, pairs_w, body, 0)
        last = iters_w - 1
        pltpu.make_async_copy(xs_hbm.at[src_a], rows_a, sem_a).wait()
        pltpu.sync_copy(rows_a, acc_s.at[dst_v.at[last]], add=True)
        pltpu.make_async_copy(src_hbm.at[wid, 0], src_b, sem_ib).wait()

        plsc.subcore_barrier()
        pltpu.sync_copy(acc_s.at[pl.ds(sid * rt, rt)],
                        out_hbm.at[cid, pl.ds(sid * rt, rt)])

    return k


def _dinv_from(degp_ref):
    deg = degp_ref[0, :, 0] + degp_ref[1, :, 0] + 1.0
    return lax.rsqrt(jnp.maximum(deg, 1e-12))


def _xs_body(x_ref, w_ref, degp_ref, o_ref):
    dinv = _dinv_from(degp_ref)
    xw = jnp.dot(x_ref[...], w_ref[...], preferred_element_type=jnp.float32)
    o_ref[...] = xw * dinv[:, None]


def _h_stats_body(nblk, n, aggp_ref, xs_ref, degp_ref, b1_ref,
                  h_ref, stats_ref, acc_ref):
    i = pl.program_id(0)
    dinv = _dinv_from(degp_ref)
    h = (aggp_ref[0] + aggp_ref[1] + xs_ref[...]) * dinv[:, None] + b1_ref[...]
    h_ref[...] = h

    @pl.when(i == 0)
    def _():
        acc_ref[...] = jnp.zeros_like(acc_ref)

    acc_ref[0:1] += jnp.sum(h, axis=0, keepdims=True)
    acc_ref[1:2] += jnp.sum(h * h, axis=0, keepdims=True)

    @pl.when(i == nblk - 1)
    def _():
        mean = acc_ref[0:1] / n
        var = acc_ref[1:2] / n - mean * mean
        stats_ref[0:1] = mean
        stats_ref[1:2] = lax.rsqrt(var + 1e-5)


def _mlp_body(h_ref, stats_ref, g_ref, be_ref, a1_ref, wp1_ref, bp1_ref,
              ap_ref, wp2_ref, bp2_ref, o_ref):
    hn = (h_ref[...] - stats_ref[0:1]) * stats_ref[1:2] * g_ref[...] + be_ref[...]
    a1 = a1_ref[0, 0]
    p = jnp.where(hn >= 0, hn, a1 * hn)
    q1 = jnp.dot(p, wp1_ref[...], preferred_element_type=jnp.float32) + bp1_ref[...]
    ap = ap_ref[0, 0]
    q1 = jnp.where(q1 >= 0, q1, ap * q1)
    o_ref[...] = jnp.dot(q1, wp2_ref[...], preferred_element_type=jnp.float32) + bp2_ref[...]


def kernel(x, edge_index, W1, b1, gamma1, beta1, a1, Wp1, bp1, ap, Wp2, bp2):
    n, d = x.shape
    e = edge_index.shape[1]
    d_pred = Wp1.shape[1]

    n_pad = -(-(n + 1) // (NS * 8)) * (NS * 8)
    rt = n_pad // NS
    nb = -(-e // EB)

    iters_d = -(-nb // NW)
    if iters_d % 2 == 0:
        iters_d += 1
    itp_d = -(-iters_d // 8) * 8
    ed_pad = iters_d * NW * EB
    dst_deg = jnp.pad(
        jnp.concatenate([edge_index[1], jnp.full((ed_pad - e,), n, jnp.int32)]
                        ).reshape(NW, iters_d, EB),
        ((0, 0), (0, itp_d - iters_d), (0, 0)), constant_values=n)

    S = -(-nb // NS)
    if S % 2 == 1:
        S += 1
    iters0 = int(0.765 * S) | 1
    iters1 = S - iters0
    itp_a = -(-(max(iters0, iters1) + 1) // 8) * 8
    ea_pad = S * NS * EB

    def plane(v, fill):
        flat = jnp.concatenate(
            [v, jnp.full((ea_pad - e,), fill, jnp.int32)])
        c0 = flat[:NS * iters0 * EB].reshape(NS, iters0, EB)
        c1 = flat[NS * iters0 * EB:].reshape(NS, iters1, EB)
        c0 = jnp.pad(c0, ((0, 0), (0, itp_a - iters0), (0, 0)),
                     constant_values=fill)
        c1 = jnp.pad(c1, ((0, 0), (0, itp_a - iters1), (0, 0)),
                     constant_values=fill)
        return jnp.stack([c0, c1], axis=1).reshape(NW, itp_a, EB)

    src_agg = plane(edge_index[0], 0)
    dst_agg = plane(edge_index[1], n)

    ones_rows = jnp.ones((EB, d), jnp.float32)
    zeros_rows = jnp.zeros((rt, d), jnp.float32)

    degp = _deg_kernel(n_pad, iters_d, itp_d, d)(dst_deg, ones_rows, zeros_rows)

    nblk = 5 if n % 5 == 0 else 1
    bn = n // nblk
    grid = (nblk,)

    xs = pl.pallas_call(
        _xs_body,
        grid=grid,
        in_specs=[
            pl.BlockSpec((bn, d), lambda i: (i, 0)),
            pl.BlockSpec((d, d), lambda i: (0, 0)),
            pl.BlockSpec((NC, bn, d), lambda i: (0, i, 0)),
        ],
        out_specs=pl.BlockSpec((bn, d), lambda i: (i, 0)),
        out_shape=jax.ShapeDtypeStruct((n, d), jnp.float32),
    )(x, W1, degp)

    aggp = _agg_kernel(n_pad, iters0, iters1, itp_a, d)(xs, src_agg, dst_agg, zeros_rows)

    h, stats = pl.pallas_call(
        functools.partial(_h_stats_body, nblk, float(n)),
        grid=grid,
        in_specs=[
            pl.BlockSpec((NC, bn, d), lambda i: (0, i, 0)),
            pl.BlockSpec((bn, d), lambda i: (i, 0)),
            pl.BlockSpec((NC, bn, d), lambda i: (0, i, 0)),
            pl.BlockSpec((1, d), lambda i: (0, 0)),
        ],
        out_specs=[
            pl.BlockSpec((bn, d), lambda i: (i, 0)),
            pl.BlockSpec((2, d), lambda i: (0, 0)),
        ],
        out_shape=[
            jax.ShapeDtypeStruct((n, d), jnp.float32),
            jax.ShapeDtypeStruct((2, d), jnp.float32),
        ],
        scratch_shapes=[pltpu.VMEM((2, d), jnp.float32)],
    )(aggp, xs, degp, b1.reshape(1, d))

    q = pl.pallas_call(
        _mlp_body,
        grid=grid,
        in_specs=[
            pl.BlockSpec((bn, d), lambda i: (i, 0)),
            pl.BlockSpec((2, d), lambda i: (0, 0)),
            pl.BlockSpec((1, d), lambda i: (0, 0)),
            pl.BlockSpec((1, d), lambda i: (0, 0)),
            pl.BlockSpec((1, 1), lambda i: (0, 0)),
            pl.BlockSpec((d, d_pred), lambda i: (0, 0)),
            pl.BlockSpec((1, d_pred), lambda i: (0, 0)),
            pl.BlockSpec((1, 1), lambda i: (0, 0)),
            pl.BlockSpec((d_pred, d), lambda i: (0, 0)),
            pl.BlockSpec((1, d), lambda i: (0, 0)),
        ],
        out_specs=pl.BlockSpec((bn, d), lambda i: (i, 0)),
        out_shape=jax.ShapeDtypeStruct((n, d), jnp.float32),
    )(h, stats, gamma1.reshape(1, d), beta1.reshape(1, d),
      a1.reshape(1, 1), Wp1, bp1.reshape(1, d_pred), ap.reshape(1, 1),
      Wp2, bp2.reshape(1, d))

    return q
